# baseline (speedup 1.0000x reference)
import jax
import jax.numpy as jnp
from jax.experimental import pallas as pl

STRUCTURE_MASK_TOKEN = 4097
STRUCTURE_PAD_TOKEN = 4100
SEQUENCE_MASK_TOKEN = 31

KS = (1832780943, 270669613)
KC = (64467757, 2916123636)

_ROT = ((13, 15, 26, 6), (17, 29, 16, 24))


def _threefry_bits(n, k0, k1):
    k0 = jnp.uint32(k0)
    k1 = jnp.uint32(k1)
    k2 = k0 ^ k1 ^ jnp.uint32(0x1BD11BDA)
    ks = (k0, k1, k2)
    x0 = jnp.full_like(n, k0)
    x1 = n + k1
    for i in range(5):
        for r in _ROT[i % 2]:
            x0 = x0 + x1
            x1 = (x1 << jnp.uint32(r)) | (x1 >> jnp.uint32(32 - r))
            x1 = x0 ^ x1
        x0 = x0 + ks[(i + 1) % 3]
        x1 = x1 + ks[(i + 2) % 3] + jnp.uint32(i + 1)
    return x0 ^ x1


def _uniform(bits):
    fb = (bits >> jnp.uint32(9)) | jnp.uint32(0x3F800000)
    return jax.lax.bitcast_convert_type(fb, jnp.float32) - jnp.float32(1.0)


_BLK = 512
_L = 8192


def _flow_kernel(structure_ref, sequence_ref, t_ref, out_struc_ref, out_seq_ref):
    structure = structure_ref[...]
    sequence = sequence_ref[...]
    t = t_ref[...]
    B, W = structure.shape

    base = pl.program_id(0) * W
    row = jax.lax.broadcasted_iota(jnp.uint32, (B, W), 0)
    col = jax.lax.broadcasted_iota(jnp.uint32, (B, W), 1)
    n = row * jnp.uint32(_L) + (col + jnp.uint32(base))

    u_seq = _uniform(_threefry_bits(n, *KS))
    u_struc = _uniform(_threefry_bits(n, *KC))

    thresh = (jnp.float32(1.0) - t)[:, :1]
    pad_mask = structure != STRUCTURE_PAD_TOKEN
    seq_mask = (u_seq < thresh) & pad_mask
    struc_mask = (u_struc < thresh) & pad_mask

    out_struc_ref[...] = jnp.where(struc_mask, STRUCTURE_MASK_TOKEN, structure)
    out_seq_ref[...] = jnp.where(seq_mask, SEQUENCE_MASK_TOKEN, sequence)


def kernel(structure, sequence, t):
    B, L = structure.shape
    grid = (L // _BLK,)
    tok_spec = pl.BlockSpec((B, _BLK), lambda i: (0, i))
    out_struc, out_seq = pl.pallas_call(
        _flow_kernel,
        grid=grid,
        in_specs=[
            tok_spec,
            tok_spec,
            pl.BlockSpec((B, 1), lambda i: (0, 0)),
        ],
        out_specs=(tok_spec, tok_spec),
        out_shape=(
            jax.ShapeDtypeStruct((B, L), structure.dtype),
            jax.ShapeDtypeStruct((B, L), sequence.dtype),
        ),
    )(structure, sequence, t[:, None])
    return (out_struc, out_seq, t)


# speedup vs baseline: 1.8730x; 1.8730x over previous
import jax
import jax.numpy as jnp
from jax.experimental import pallas as pl

STRUCTURE_MASK_TOKEN = 4097
STRUCTURE_PAD_TOKEN = 4100
SEQUENCE_MASK_TOKEN = 31

KS = (1832780943, 270669613)
KC = (64467757, 2916123636)

_ROT = ((13, 15, 26, 6), (17, 29, 16, 24))


def _threefry_bits(n, k0, k1):
    k0 = jnp.uint32(k0)
    k1 = jnp.uint32(k1)
    k2 = k0 ^ k1 ^ jnp.uint32(0x1BD11BDA)
    ks = (k0, k1, k2)
    x0 = jnp.full_like(n, k0)
    x1 = n + k1
    for i in range(5):
        for r in _ROT[i % 2]:
            x0 = x0 + x1
            x1 = (x1 << jnp.uint32(r)) | (x1 >> jnp.uint32(32 - r))
            x1 = x0 ^ x1
        x0 = x0 + ks[(i + 1) % 3]
        x1 = x1 + ks[(i + 2) % 3] + jnp.uint32(i + 1)
    return x0 ^ x1


def _uniform(bits):
    fb = (bits >> jnp.uint32(9)) | jnp.uint32(0x3F800000)
    return jax.lax.bitcast_convert_type(fb, jnp.float32) - jnp.float32(1.0)


_BLK = 512


def _flow_kernel(structure_ref, sequence_ref, t_ref, out_struc_ref, out_seq_ref):
    B, L = structure_ref.shape
    thresh = (jnp.float32(1.0) - t_ref[...])[:, :1]

    for i in range(L // _BLK):
        sl = pl.ds(i * _BLK, _BLK)
        structure = structure_ref[:, sl]
        sequence = sequence_ref[:, sl]

        row = jax.lax.broadcasted_iota(jnp.uint32, (B, _BLK), 0)
        col = jax.lax.broadcasted_iota(jnp.uint32, (B, _BLK), 1)
        n = row * jnp.uint32(L) + (col + jnp.uint32(i * _BLK))

        u_seq = _uniform(_threefry_bits(n, *KS))
        u_struc = _uniform(_threefry_bits(n, *KC))

        pad_mask = structure != STRUCTURE_PAD_TOKEN
        seq_mask = (u_seq < thresh) & pad_mask
        struc_mask = (u_struc < thresh) & pad_mask

        out_struc_ref[:, sl] = jnp.where(struc_mask, STRUCTURE_MASK_TOKEN, structure)
        out_seq_ref[:, sl] = jnp.where(seq_mask, SEQUENCE_MASK_TOKEN, sequence)


def kernel(structure, sequence, t):
    B, L = structure.shape
    out_struc, out_seq = pl.pallas_call(
        _flow_kernel,
        out_shape=(
            jax.ShapeDtypeStruct((B, L), structure.dtype),
            jax.ShapeDtypeStruct((B, L), sequence.dtype),
        ),
    )(structure, sequence, t[:, None])
    return (out_struc, out_seq, t)
